# baseline (speedup 1.0000x reference)
import functools
import math

import jax
import jax.numpy as jnp
from jax.experimental import pallas as pl
from jax.experimental.pallas import tpu as pltpu
from jax.experimental.pallas import tpu_sc as plsc

_SIZE = 32768
_N = 2048
_SMOOTHING = 0.1
_CONF = 1.0 - _SMOOTHING
_EPS = _SMOOTHING / (_SIZE - 2)
_C_CONST = _CONF * math.log(_CONF) + _SMOOTHING * math.log(_EPS)
_CBLK = 2048

_NC = 1
_NW = _NC * 16
_RPW = _N // _NW
_L = 16


def _tc_body(t_ref, x_ref, out_ref):
    j = pl.program_id(0)
    t = t_ref[...]
    mrow = (t != 0).astype(jnp.float32)
    xb = x_ref[...]
    rs = jnp.sum(xb, axis=1, keepdims=True)
    acc = -_EPS * jnp.sum(mrow * rs)

    @pl.when(j == 0)
    def _init():
        k_valid = jnp.sum(mrow)
        x0 = jnp.sum(mrow * xb[:, 0:1])
        out_ref[0, 0] = _C_CONST * k_valid + _EPS * x0

    out_ref[0, 0] += acc


_WAVE = 32


def _sc_body(x_hbm, t_hbm, out_hbm, t_v, tiles_v, junk_v, acc_v, sem):
    cid = jax.lax.axis_index("c")
    sid = jax.lax.axis_index("s")
    wid = sid * _NC + cid
    base = wid * _RPW
    pltpu.sync_copy(t_hbm.at[pl.ds(base, _RPW)], t_v)
    lanes = jax.lax.iota(jnp.int32, _L)
    acc = jnp.zeros((_L,), jnp.float32)
    for w in range(_RPW // _WAVE):
        trs = []
        for kk in range(_WAVE // _L):
            tw = t_v[pl.ds(w * _WAVE + kk * _L, _L)]
            trs.extend(tw[q] for q in range(_L))
        copies = []
        for q in range(_WAVE):
            r = w * _WAVE + q
            tr = trs[q]
            c0 = pl.multiple_of((tr >> 7) << 7, 128)
            row0 = pl.multiple_of(base + (r // 8) * 8, 8)
            cp = pltpu.make_async_copy(
                x_hbm.at[pl.ds(row0, 8), pl.ds(c0, 128)], tiles_v.at[q], sem
            )
            cp.start()
            copies.append(cp)
        for cp in copies:
            cp.wait()
        for q in range(_WAVE):
            r = w * _WAVE + q
            tr = trs[q]
            cs = (tr >> 4 << 4) & 127
            v16 = tiles_v[q, r % 8, pl.ds(cs, _L)]
            lane_sel = jnp.where(tr != 0, tr & (_L - 1), jnp.int32(-1))
            acc = acc + jnp.where(lanes == lane_sel, v16, 0.0)
    prev = None
    for c in range(64):
        cp = pltpu.make_async_copy(
            x_hbm.at[
                pl.ds(base + 8 * (c // 16), 8), pl.ds((c % 16) * 2048, 2048)
            ],
            junk_v.at[c % 2],
            sem,
        )
        cp.start()
        if prev is not None:
            prev.wait()
        prev = cp
    prev.wait()
    acc_v[...] = acc
    pltpu.sync_copy(acc_v, out_hbm.at[wid])


_sc_gather = functools.partial(
    pl.kernel,
    out_type=jax.ShapeDtypeStruct((_NW, _L), jnp.float32),
    mesh=plsc.VectorSubcoreMesh(
        core_axis_name="c", subcore_axis_name="s", num_cores=_NC
    ),
    scratch_types=[
        pltpu.VMEM((_RPW,), jnp.int32),
        pltpu.VMEM((_WAVE, 8, 128), jnp.float32),
        pltpu.VMEM((2, 8, 2048), jnp.float32),
        pltpu.VMEM((_L,), jnp.float32),
        pltpu.SemaphoreType.DMA,
    ],
)(_sc_body)


def kernel(x, target):
    t32 = target.astype(jnp.int32)
    t2d = t32.reshape(_N, 1)
    sc_out = _sc_gather(x, t32)
    tc_out = pl.pallas_call(
        _tc_body,
        grid=(_SIZE // _CBLK,),
        in_specs=[
            pl.BlockSpec((_N, 1), lambda j: (0, 0)),
            pl.BlockSpec((_N, _CBLK), lambda j: (0, j)),
        ],
        out_specs=pl.BlockSpec(
            (1, 1), lambda j: (0, 0), memory_space=pltpu.SMEM
        ),
        out_shape=jax.ShapeDtypeStruct((1, 1), jnp.float32),
    )(t2d, x)
    return tc_out[0, 0] - (_CONF - _EPS) * jnp.sum(sc_out)


# speedup vs baseline: 1.1961x; 1.1961x over previous
import functools
import math

import jax
import jax.numpy as jnp
from jax.experimental import pallas as pl
from jax.experimental.pallas import tpu as pltpu
from jax.experimental.pallas import tpu_sc as plsc

_SIZE = 32768
_N = 2048
_SMOOTHING = 0.1
_CONF = 1.0 - _SMOOTHING
_EPS = _SMOOTHING / (_SIZE - 2)
_C_CONST = _CONF * math.log(_CONF) + _SMOOTHING * math.log(_EPS)
_CBLK = 1024

_NC = 1
_NW = _NC * 16
_RPW = _N // _NW
_L = 16


def _tc_body(t_ref, x_ref, out_ref):
    j = pl.program_id(0)
    t = t_ref[...]
    mrow = (t != 0).astype(jnp.float32)
    xb = x_ref[...]
    rs = jnp.sum(xb, axis=1, keepdims=True)
    acc = -_EPS * jnp.sum(mrow * rs)

    @pl.when(j == 0)
    def _init():
        k_valid = jnp.sum(mrow)
        x0 = jnp.sum(mrow * xb[:, 0:1])
        out_ref[0, 0] = _C_CONST * k_valid + _EPS * x0

    out_ref[0, 0] += acc


_WAVE = 32


def _sc_body(x_hbm, t_hbm, out_hbm, t_v, tiles_v, acc_v, sem):
    cid = jax.lax.axis_index("c")
    sid = jax.lax.axis_index("s")
    wid = sid * _NC + cid
    base = wid * _RPW
    pltpu.sync_copy(t_hbm.at[pl.ds(base, _RPW)], t_v)
    lanes = jax.lax.iota(jnp.int32, _L)
    acc = jnp.zeros((_L,), jnp.float32)
    for w in range(_RPW // _WAVE):
        trs = []
        for kk in range(_WAVE // _L):
            tw = t_v[pl.ds(w * _WAVE + kk * _L, _L)]
            trs.extend(tw[q] for q in range(_L))
        copies = []
        for q in range(_WAVE):
            r = w * _WAVE + q
            tr = trs[q]
            c0 = pl.multiple_of((tr >> 7) << 7, 128)
            row0 = pl.multiple_of(base + (r // 8) * 8, 8)
            cp = pltpu.make_async_copy(
                x_hbm.at[pl.ds(row0, 8), pl.ds(c0, 128)], tiles_v.at[q], sem
            )
            cp.start()
            copies.append(cp)
        for cp in copies:
            cp.wait()
        for q in range(_WAVE):
            r = w * _WAVE + q
            tr = trs[q]
            cs = (tr >> 4 << 4) & 127
            v16 = tiles_v[q, r % 8, pl.ds(cs, _L)]
            lane_sel = jnp.where(tr != 0, tr & (_L - 1), jnp.int32(-1))
            acc = acc + jnp.where(lanes == lane_sel, v16, 0.0)
    acc_v[...] = acc
    pltpu.sync_copy(acc_v, out_hbm.at[wid])


_sc_gather = functools.partial(
    pl.kernel,
    out_type=jax.ShapeDtypeStruct((_NW, _L), jnp.float32),
    mesh=plsc.VectorSubcoreMesh(
        core_axis_name="c", subcore_axis_name="s", num_cores=_NC
    ),
    scratch_types=[
        pltpu.VMEM((_RPW,), jnp.int32),
        pltpu.VMEM((_WAVE, 8, 128), jnp.float32),
        pltpu.VMEM((_L,), jnp.float32),
        pltpu.SemaphoreType.DMA,
    ],
)(_sc_body)


def kernel(x, target):
    t32 = target.astype(jnp.int32)
    t2d = t32.reshape(_N, 1)
    sc_out = _sc_gather(x, t32)
    tc_out = pl.pallas_call(
        _tc_body,
        grid=(_SIZE // _CBLK,),
        in_specs=[
            pl.BlockSpec((_N, 1), lambda j: (0, 0)),
            pl.BlockSpec((_N, _CBLK), lambda j: (0, j)),
        ],
        out_specs=pl.BlockSpec(
            (1, 1), lambda j: (0, 0), memory_space=pltpu.SMEM
        ),
        out_shape=jax.ShapeDtypeStruct((1, 1), jnp.float32),
    )(t2d, x)
    return tc_out[0, 0] - (_CONF - _EPS) * jnp.sum(sc_out)


# speedup vs baseline: 1.2041x; 1.0067x over previous
import functools
import math

import jax
import jax.numpy as jnp
from jax.experimental import pallas as pl
from jax.experimental.pallas import tpu as pltpu
from jax.experimental.pallas import tpu_sc as plsc

_SIZE = 32768
_N = 2048
_SMOOTHING = 0.1
_CONF = 1.0 - _SMOOTHING
_EPS = _SMOOTHING / (_SIZE - 2)
_C_CONST = _CONF * math.log(_CONF) + _SMOOTHING * math.log(_EPS)
_CBLK = 2048

_NC = 1
_NW = _NC * 16
_RPW = _N // _NW
_L = 16


def _tc_body(t_ref, x_ref, out_ref):
    j = pl.program_id(0)
    t = t_ref[...]
    mrow = (t != 0).astype(jnp.float32)
    xb = x_ref[...]
    rs = jnp.sum(xb, axis=1, keepdims=True)
    acc = -_EPS * jnp.sum(mrow * rs)

    @pl.when(j == 0)
    def _init():
        k_valid = jnp.sum(mrow)
        x0 = jnp.sum(mrow * xb[:, 0:1])
        out_ref[0, 0] = _C_CONST * k_valid + _EPS * x0

    out_ref[0, 0] += acc


_WAVE = 64


def _sc_body(x_hbm, t_hbm, out_hbm, t_v, tiles_v, acc_v, sem):
    cid = jax.lax.axis_index("c")
    sid = jax.lax.axis_index("s")
    wid = sid * _NC + cid
    base = wid * _RPW
    pltpu.sync_copy(t_hbm.at[pl.ds(base, _RPW)], t_v)
    lanes = jax.lax.iota(jnp.int32, _L)
    acc = jnp.zeros((_L,), jnp.float32)
    for w in range(_RPW // _WAVE):
        trs = []
        for kk in range(_WAVE // _L):
            tw = t_v[pl.ds(w * _WAVE + kk * _L, _L)]
            trs.extend(tw[q] for q in range(_L))
        copies = []
        for q in range(_WAVE):
            r = w * _WAVE + q
            tr = trs[q]
            c0 = pl.multiple_of((tr >> 7) << 7, 128)
            row0 = pl.multiple_of(base + (r // 8) * 8, 8)
            cp = pltpu.make_async_copy(
                x_hbm.at[pl.ds(row0, 8), pl.ds(c0, 128)], tiles_v.at[q], sem
            )
            cp.start()
            copies.append(cp)
        for cp in copies:
            cp.wait()
        for q in range(_WAVE):
            r = w * _WAVE + q
            tr = trs[q]
            cs = (tr >> 4 << 4) & 127
            v16 = tiles_v[q, r % 8, pl.ds(cs, _L)]
            lane_sel = jnp.where(tr != 0, tr & (_L - 1), jnp.int32(-1))
            acc = acc + jnp.where(lanes == lane_sel, v16, 0.0)
    acc_v[...] = acc
    pltpu.sync_copy(acc_v, out_hbm.at[wid])


_sc_gather = functools.partial(
    pl.kernel,
    out_type=jax.ShapeDtypeStruct((_NW, _L), jnp.float32),
    mesh=plsc.VectorSubcoreMesh(
        core_axis_name="c", subcore_axis_name="s", num_cores=_NC
    ),
    scratch_types=[
        pltpu.VMEM((_RPW,), jnp.int32),
        pltpu.VMEM((_WAVE, 8, 128), jnp.float32),
        pltpu.VMEM((_L,), jnp.float32),
        pltpu.SemaphoreType.DMA,
    ],
)(_sc_body)


def kernel(x, target):
    t32 = target.astype(jnp.int32)
    t2d = t32.reshape(_N, 1)
    sc_out = _sc_gather(x, t32)
    tc_out = pl.pallas_call(
        _tc_body,
        grid=(_SIZE // _CBLK,),
        in_specs=[
            pl.BlockSpec((_N, 1), lambda j: (0, 0)),
            pl.BlockSpec((_N, _CBLK), lambda j: (0, j)),
        ],
        out_specs=pl.BlockSpec(
            (1, 1), lambda j: (0, 0), memory_space=pltpu.SMEM
        ),
        out_shape=jax.ShapeDtypeStruct((1, 1), jnp.float32),
    )(t2d, x)
    return tc_out[0, 0] - (_CONF - _EPS) * jnp.sum(sc_out)


# speedup vs baseline: 1.2334x; 1.0244x over previous
import functools
import math

import jax
import jax.numpy as jnp
from jax.experimental import pallas as pl
from jax.experimental.pallas import tpu as pltpu
from jax.experimental.pallas import tpu_sc as plsc

_SIZE = 32768
_N = 2048
_SMOOTHING = 0.1
_CONF = 1.0 - _SMOOTHING
_EPS = _SMOOTHING / (_SIZE - 2)
_C_CONST = _CONF * math.log(_CONF) + _SMOOTHING * math.log(_EPS)
_CBLK = 2048

_NC = 1
_NW = _NC * 16
_RPW = _N // _NW
_L = 16


def _tc_body(t_ref, x_ref, out_ref):
    j = pl.program_id(0)
    t = t_ref[...]
    m = (t != 0).astype(jnp.float32)
    xb = x_ref[...]
    rs = jnp.sum(xb, axis=1, keepdims=True)
    acc = -_EPS * jnp.dot(m, rs, preferred_element_type=jnp.float32)[0, 0]

    @pl.when(j == 0)
    def _init():
        k_valid = jnp.sum(m)
        x0 = jnp.dot(m, xb[:, 0:1], preferred_element_type=jnp.float32)[0, 0]
        out_ref[0, 0] = _C_CONST * k_valid + _EPS * x0

    out_ref[0, 0] += acc


_WAVE = 64


def _sc_body(x_hbm, t_hbm, out_hbm, t_v, tiles_v, acc_v, sem):
    cid = jax.lax.axis_index("c")
    sid = jax.lax.axis_index("s")
    wid = sid * _NC + cid
    base = wid * _RPW
    pltpu.sync_copy(t_hbm.at[pl.ds(base, _RPW)], t_v)
    lanes = jax.lax.iota(jnp.int32, _L)
    acc = jnp.zeros((_L,), jnp.float32)
    for w in range(_RPW // _WAVE):
        trs = []
        for kk in range(_WAVE // _L):
            tw = t_v[pl.ds(w * _WAVE + kk * _L, _L)]
            trs.extend(tw[q] for q in range(_L))
        copies = []
        for q in range(_WAVE):
            r = w * _WAVE + q
            tr = trs[q]
            c0 = pl.multiple_of((tr >> 7) << 7, 128)
            row0 = pl.multiple_of(base + (r // 8) * 8, 8)
            cp = pltpu.make_async_copy(
                x_hbm.at[pl.ds(row0, 8), pl.ds(c0, 128)], tiles_v.at[q], sem
            )
            cp.start()
            copies.append(cp)
        for cp in copies:
            cp.wait()
        for q in range(_WAVE):
            r = w * _WAVE + q
            tr = trs[q]
            cs = (tr >> 4 << 4) & 127
            v16 = tiles_v[q, r % 8, pl.ds(cs, _L)]
            lane_sel = jnp.where(tr != 0, tr & (_L - 1), jnp.int32(-1))
            acc = acc + jnp.where(lanes == lane_sel, v16, 0.0)
    acc_v[...] = acc
    pltpu.sync_copy(acc_v, out_hbm.at[wid])


_sc_gather = functools.partial(
    pl.kernel,
    out_type=jax.ShapeDtypeStruct((_NW, _L), jnp.float32),
    mesh=plsc.VectorSubcoreMesh(
        core_axis_name="c", subcore_axis_name="s", num_cores=_NC
    ),
    scratch_types=[
        pltpu.VMEM((_RPW,), jnp.int32),
        pltpu.VMEM((_WAVE, 8, 128), jnp.float32),
        pltpu.VMEM((_L,), jnp.float32),
        pltpu.SemaphoreType.DMA,
    ],
)(_sc_body)


def kernel(x, target):
    t32 = target.astype(jnp.int32)
    t2d = t32.reshape(1, _N)
    sc_out = _sc_gather(x, t32)
    tc_out = pl.pallas_call(
        _tc_body,
        grid=(_SIZE // _CBLK,),
        in_specs=[
            pl.BlockSpec((1, _N), lambda j: (0, 0)),
            pl.BlockSpec((_N, _CBLK), lambda j: (0, j)),
        ],
        out_specs=pl.BlockSpec(
            (1, 1), lambda j: (0, 0), memory_space=pltpu.SMEM
        ),
        out_shape=jax.ShapeDtypeStruct((1, 1), jnp.float32),
    )(t2d, x)
    return tc_out[0, 0] - (_CONF - _EPS) * jnp.sum(sc_out)
